# baseline (speedup 1.0000x reference)
import functools

import jax
import jax.numpy as jnp
from jax import lax
from jax.experimental import pallas as pl
from jax.experimental.pallas import tpu as pltpu
from jax.experimental.pallas import tpu_sc as plsc

_N = 10000
_E = 320000
_IN = 128
_DM = 32
_DC = 80
_DP = 128
_NC, _NS, _L = 2, 16, 16
_NW = _NC * _NS
_ET = _E // _NW
_K = 40
_NCHUNK = _ET // _K
_NSUP = 10
_SUB = _NCHUNK // _NSUP
_NB = 8
_RPT = _N // _NS
_ZR = 25
_BR = 2000


def _mesh():
    return plsc.VectorSubcoreMesh(core_axis_name="c", subcore_axis_name="s")


def _sc_degree(dst4):
    @functools.partial(
        pl.kernel, mesh=_mesh(),
        out_type=jax.ShapeDtypeStruct((_NC * _N,), jnp.float32),
        scratch_types=[
            pltpu.VMEM((_SUB, _K), jnp.int32),
            pltpu.VMEM((_K,), jnp.float32),
            pltpu.VMEM((640,), jnp.float32),
            pltpu.VMEM_SHARED((_N,), jnp.float32),
        ],
    )
    def deg_kernel(dst_hbm, out_hbm, didx, ones, zbuf, acc):
        c = lax.axis_index("c")
        s = lax.axis_index("s")
        wid = c * _NS + s

        @pl.loop(0, _K // _L)
        def _fill_ones(i):
            ones[pl.ds(i * _L, _L)] = jnp.full((_L,), 1.0, jnp.float32)

        @pl.loop(0, 640 // _L)
        def _fill_zero(i):
            zbuf[pl.ds(i * _L, _L)] = jnp.zeros((_L,), jnp.float32)

        @pl.loop(s, _N // 640, step=_NS)
        def _zero(j):
            pltpu.sync_copy(zbuf, acc.at[pl.ds(j * 640, 640)])

        @pl.when(s == 0)
        def _zero_tail():
            pltpu.sync_copy(zbuf.at[pl.ds(0, 400)], acc.at[pl.ds(9600, 400)])

        plsc.subcore_barrier()

        @pl.loop(0, _NSUP)
        def _super(u):
            pltpu.sync_copy(dst_hbm.at[wid, u], didx)

            @pl.loop(0, _SUB)
            def _edges(j):
                pltpu.sync_copy(ones, acc.at[didx.at[j]], add=True)

        plsc.subcore_barrier()

        @pl.loop(s, _N // 80, step=_NS)
        def _wb(j):
            pltpu.sync_copy(acc.at[pl.ds(j * 80, 80)], zbuf.at[pl.ds(0, 80)])
            pltpu.sync_copy(zbuf.at[pl.ds(0, 80)],
                            out_hbm.at[pl.ds(c * _N + j * 80, 80)])

    return deg_kernel(dst4)


def _sc_propagate(src4, dst4, g, zrows):
    @functools.partial(
        pl.kernel, mesh=_mesh(),
        out_type=jax.ShapeDtypeStruct((_NC, _N, _DP), jnp.float32),
        scratch_types=[
            pltpu.VMEM((_SUB, _K), jnp.int32),
            pltpu.VMEM((_SUB, _K), jnp.int32),
            pltpu.VMEM((_NB, _K, _DP), jnp.float32),
            pltpu.VMEM_SHARED((_N, _DP), jnp.float32),
            pltpu.SemaphoreType.DMA,
            pltpu.SemaphoreType.DMA,
        ],
    )
    def prop_kernel(src_hbm, dst_hbm, g_hbm, z_hbm, out_hbm, sidx, didx,
                    rows, acc, gsem, ssem):
        c = lax.axis_index("c")
        s = lax.axis_index("s")
        wid = c * _NS + s

        def _drain(ref, sem):
            pltpu.make_async_copy(g_hbm.at[pl.ds(0, _K)], ref, sem).wait()

        @pl.loop(s, _N // 80, step=_NS)
        def _zero(j):
            pltpu.sync_copy(z_hbm.at[pl.ds(j * 80, 80)],
                            acc.at[pl.ds(j * 80, 80)])

        plsc.subcore_barrier()

        @pl.loop(0, _NSUP)
        def _super(u):
            pltpu.sync_copy(src_hbm.at[wid, u], sidx)
            pltpu.sync_copy(dst_hbm.at[wid, u], didx)
            for j in range(_NB - 1):
                pltpu.async_copy(g_hbm.at[sidx.at[j]], rows.at[j], gsem)

            for j in range(_SUB):
                _drain(rows.at[j % _NB], gsem)
                if j >= 1:
                    _drain(rows.at[(j - 1) % _NB], ssem)
                if j < _SUB - (_NB - 1):
                    pltpu.async_copy(g_hbm.at[sidx.at[j + _NB - 1]],
                                     rows.at[(j + _NB - 1) % _NB], gsem)
                pltpu.async_copy(rows.at[j % _NB], acc.at[didx.at[j]], ssem,
                                 add=True)

            _drain(rows.at[0], ssem)

        plsc.subcore_barrier()

        @pl.loop(s, _N // 80, step=_NS)
        def _wb(j):
            pltpu.sync_copy(acc.at[pl.ds(j * 80, 80)],
                            out_hbm.at[c, pl.ds(j * 80, 80)])

    return prop_kernel(src4, dst4, g, zrows)


def _dinv_block(c0_ref, c1_ref):
    deg = c0_ref[...] + c1_ref[...] + 1.0
    return lax.rsqrt(deg)


def _tc_matmul(x, w):
    def body(x_ref, w_ref, o_ref):
        o_ref[...] = jnp.dot(x_ref[...], w_ref[...],
                             preferred_element_type=jnp.float32)

    return pl.pallas_call(
        body,
        grid=(_N // _BR,),
        in_specs=[pl.BlockSpec((_BR, _IN), lambda i: (i, 0)),
                  pl.BlockSpec((_IN, _DM), lambda i: (0, 0))],
        out_specs=pl.BlockSpec((_BR, _DM), lambda i: (i, 0)),
        out_shape=jax.ShapeDtypeStruct((_N, _DM), jnp.float32),
    )(x, w)


def _tc_scale(c0, c1, h):
    def body(c0_ref, c1_ref, h_ref, o_ref):
        g = _dinv_block(c0_ref, c1_ref) * h_ref[...]
        o_ref[...] = jnp.concatenate(
            [g, jnp.zeros((_BR, _DP - _DM), jnp.float32)], axis=1)

    return pl.pallas_call(
        body,
        grid=(_N // _BR,),
        in_specs=[pl.BlockSpec((_BR, 1), lambda i: (i, 0)),
                  pl.BlockSpec((_BR, 1), lambda i: (i, 0)),
                  pl.BlockSpec((_BR, _DM), lambda i: (i, 0))],
        out_specs=pl.BlockSpec((_BR, _DP), lambda i: (i, 0)),
        out_shape=jax.ShapeDtypeStruct((_N, _DP), jnp.float32),
    )(c0, c1, h)


def _tc_mid(s1, g1, c0, c1, b1, wcat):
    def body(s_ref, g_ref, c0_ref, c1_ref, b_ref, w_ref, o_ref):
        dinv = _dinv_block(c0_ref, c1_ref)
        tot = s_ref[0, :, :_DM] + s_ref[1, :, :_DM] + g_ref[:, :_DM]
        h = jnp.maximum(dinv * tot + b_ref[...], 0.0)
        o_ref[...] = dinv * jnp.dot(h, w_ref[...],
                                    preferred_element_type=jnp.float32)

    return pl.pallas_call(
        body,
        grid=(_N // _BR,),
        in_specs=[pl.BlockSpec((_NC, _BR, _DP), lambda i: (0, i, 0)),
                  pl.BlockSpec((_BR, _DP), lambda i: (i, 0)),
                  pl.BlockSpec((_BR, 1), lambda i: (i, 0)),
                  pl.BlockSpec((_BR, 1), lambda i: (i, 0)),
                  pl.BlockSpec((1, _DM), lambda i: (0, 0)),
                  pl.BlockSpec((_DM, _DP), lambda i: (0, 0))],
        out_specs=pl.BlockSpec((_BR, _DP), lambda i: (i, 0)),
        out_shape=jax.ShapeDtypeStruct((_N, _DP), jnp.float32),
    )(s1, g1, c0, c1, b1, wcat)


def _tc_final(s2, g2, c0, c1, bcat):
    def body(s_ref, g_ref, c0_ref, c1_ref, b_ref, mu_ref, sg_ref):
        tot = s_ref[0, :, :_DC] + s_ref[1, :, :_DC] + g_ref[:, :_DC]
        out = _dinv_block(c0_ref, c1_ref) * tot + b_ref[...]
        mu_ref[...] = out[:, :64]
        sg_ref[...] = out[:, 64:65]

    return pl.pallas_call(
        body,
        grid=(_N // _BR,),
        in_specs=[pl.BlockSpec((_NC, _BR, _DP), lambda i: (0, i, 0)),
                  pl.BlockSpec((_BR, _DP), lambda i: (i, 0)),
                  pl.BlockSpec((_BR, 1), lambda i: (i, 0)),
                  pl.BlockSpec((_BR, 1), lambda i: (i, 0)),
                  pl.BlockSpec((1, _DC), lambda i: (0, 0))],
        out_specs=[pl.BlockSpec((_BR, 64), lambda i: (i, 0)),
                   pl.BlockSpec((_BR, 1), lambda i: (i, 0))],
        out_shape=[jax.ShapeDtypeStruct((_N, 64), jnp.float32),
                   jax.ShapeDtypeStruct((_N, 1), jnp.float32)],
    )(s2, g2, c0, c1, bcat)


def kernel(X, graph, W1, b1, W_mu, b_mu, W_sig, b_sig):
    graph = graph.astype(jnp.int32)
    src4 = graph[0].reshape(_NW, _NSUP, _SUB, _K)
    dst4 = graph[1].reshape(_NW, _NSUP, _SUB, _K)
    wcat = jnp.concatenate(
        [W_mu, W_sig, jnp.zeros((_DM, _DP - 65), jnp.float32)], axis=1)
    bcat = jnp.concatenate(
        [b_mu, b_sig, jnp.zeros((_DC - 65,), jnp.float32)]).reshape(1, _DC)
    b1r = b1.reshape(1, _DM)

    h1 = _tc_matmul(X, W1)
    cnt = _sc_degree(dst4)
    c0 = cnt[:_N].reshape(_N, 1)
    c1 = cnt[_N:].reshape(_N, 1)
    g1 = _tc_scale(c0, c1, h1)
    zrows = jnp.zeros((_N, _DP), jnp.float32)
    s1 = _sc_propagate(src4, dst4, g1, zrows)
    g2 = _tc_mid(s1, g1, c0, c1, b1r, wcat)
    s2 = _sc_propagate(src4, dst4, g2, zrows)
    mus, logsig = _tc_final(s2, g2, c0, c1, bcat)
    return (mus, logsig)


# speedup vs baseline: 1.1322x; 1.1322x over previous
import functools

import jax
import jax.numpy as jnp
from jax import lax
from jax.experimental import pallas as pl
from jax.experimental.pallas import tpu as pltpu
from jax.experimental.pallas import tpu_sc as plsc

_N = 10000
_E = 320000
_IN = 128
_DM = 32
_DC = 80
_DP = 128
_NC, _NS, _L = 2, 16, 16
_NW = _NC * _NS
_ET = _E // _NW
_K = 80
_NCHUNK = _ET // _K
_NSUP = 5
_SUB = _NCHUNK // _NSUP
_NB = 4
_RPT = _N // _NS
_ZR = 25
_BR = 2000


def _mesh():
    return plsc.VectorSubcoreMesh(core_axis_name="c", subcore_axis_name="s")


def _sc_degree(dst4):
    @functools.partial(
        pl.kernel, mesh=_mesh(),
        out_type=jax.ShapeDtypeStruct((_NC * _N,), jnp.float32),
        scratch_types=[
            pltpu.VMEM((_SUB, _K), jnp.int32),
            pltpu.VMEM((_K,), jnp.float32),
            pltpu.VMEM((640,), jnp.float32),
            pltpu.VMEM_SHARED((_N,), jnp.float32),
        ],
    )
    def deg_kernel(dst_hbm, out_hbm, didx, ones, zbuf, acc):
        c = lax.axis_index("c")
        s = lax.axis_index("s")
        wid = c * _NS + s

        @pl.loop(0, _K // _L)
        def _fill_ones(i):
            ones[pl.ds(i * _L, _L)] = jnp.full((_L,), 1.0, jnp.float32)

        @pl.loop(0, 640 // _L)
        def _fill_zero(i):
            zbuf[pl.ds(i * _L, _L)] = jnp.zeros((_L,), jnp.float32)

        @pl.loop(s, _N // 640, step=_NS)
        def _zero(j):
            pltpu.sync_copy(zbuf, acc.at[pl.ds(j * 640, 640)])

        @pl.when(s == 0)
        def _zero_tail():
            pltpu.sync_copy(zbuf.at[pl.ds(0, 400)], acc.at[pl.ds(9600, 400)])

        plsc.subcore_barrier()

        @pl.loop(0, _NSUP)
        def _super(u):
            pltpu.sync_copy(dst_hbm.at[wid, u], didx)

            @pl.loop(0, _SUB)
            def _edges(j):
                pltpu.sync_copy(ones, acc.at[didx.at[j]], add=True)

        plsc.subcore_barrier()

        @pl.loop(s, _N // 80, step=_NS)
        def _wb(j):
            pltpu.sync_copy(acc.at[pl.ds(j * 80, 80)], zbuf.at[pl.ds(0, 80)])
            pltpu.sync_copy(zbuf.at[pl.ds(0, 80)],
                            out_hbm.at[pl.ds(c * _N + j * 80, 80)])

    return deg_kernel(dst4)


def _sc_propagate(src4, dst4, g, zrows):
    @functools.partial(
        pl.kernel, mesh=_mesh(),
        out_type=jax.ShapeDtypeStruct((_NC, _N, _DP), jnp.float32),
        scratch_types=[
            pltpu.VMEM((_SUB, _K), jnp.int32),
            pltpu.VMEM((_SUB, _K), jnp.int32),
            pltpu.VMEM((_NB, _K, _DP), jnp.float32),
            pltpu.VMEM_SHARED((_N, _DP), jnp.float32),
            pltpu.SemaphoreType.DMA,
            pltpu.SemaphoreType.DMA,
        ],
    )
    def prop_kernel(src_hbm, dst_hbm, g_hbm, z_hbm, out_hbm, sidx, didx,
                    rows, acc, gsem, ssem):
        c = lax.axis_index("c")
        s = lax.axis_index("s")
        wid = c * _NS + s

        def _drain(ref, sem):
            pltpu.make_async_copy(g_hbm.at[pl.ds(0, _K)], ref, sem).wait()

        @pl.loop(s, _N // 80, step=_NS)
        def _zero(j):
            pltpu.sync_copy(z_hbm.at[pl.ds(j * 80, 80)],
                            acc.at[pl.ds(j * 80, 80)])

        plsc.subcore_barrier()

        @pl.loop(0, _NSUP)
        def _super(u):
            pltpu.sync_copy(src_hbm.at[wid, u], sidx)
            pltpu.sync_copy(dst_hbm.at[wid, u], didx)
            for j in range(_NB - 1):
                pltpu.async_copy(g_hbm.at[sidx.at[j]], rows.at[j], gsem)

            for j in range(_SUB):
                _drain(rows.at[j % _NB], gsem)
                if j >= 1:
                    _drain(rows.at[(j - 1) % _NB], ssem)
                if j < _SUB - (_NB - 1):
                    pltpu.async_copy(g_hbm.at[sidx.at[j + _NB - 1]],
                                     rows.at[(j + _NB - 1) % _NB], gsem)
                pltpu.async_copy(rows.at[j % _NB], acc.at[didx.at[j]], ssem,
                                 add=True)

            _drain(rows.at[0], ssem)

        plsc.subcore_barrier()

        @pl.loop(s, _N // 80, step=_NS)
        def _wb(j):
            pltpu.sync_copy(acc.at[pl.ds(j * 80, 80)],
                            out_hbm.at[c, pl.ds(j * 80, 80)])

    return prop_kernel(src4, dst4, g, zrows)


def _dinv_block(c0_ref, c1_ref):
    deg = c0_ref[...] + c1_ref[...] + 1.0
    return lax.rsqrt(deg)


def _tc_matmul(x, w):
    def body(x_ref, w_ref, o_ref):
        o_ref[...] = jnp.dot(x_ref[...], w_ref[...],
                             preferred_element_type=jnp.float32)

    return pl.pallas_call(
        body,
        grid=(_N // _BR,),
        in_specs=[pl.BlockSpec((_BR, _IN), lambda i: (i, 0)),
                  pl.BlockSpec((_IN, _DM), lambda i: (0, 0))],
        out_specs=pl.BlockSpec((_BR, _DM), lambda i: (i, 0)),
        out_shape=jax.ShapeDtypeStruct((_N, _DM), jnp.float32),
    )(x, w)


def _tc_scale(c0, c1, h):
    def body(c0_ref, c1_ref, h_ref, o_ref):
        g = _dinv_block(c0_ref, c1_ref) * h_ref[...]
        o_ref[...] = jnp.concatenate(
            [g, jnp.zeros((_BR, _DP - _DM), jnp.float32)], axis=1)

    return pl.pallas_call(
        body,
        grid=(_N // _BR,),
        in_specs=[pl.BlockSpec((_BR, 1), lambda i: (i, 0)),
                  pl.BlockSpec((_BR, 1), lambda i: (i, 0)),
                  pl.BlockSpec((_BR, _DM), lambda i: (i, 0))],
        out_specs=pl.BlockSpec((_BR, _DP), lambda i: (i, 0)),
        out_shape=jax.ShapeDtypeStruct((_N, _DP), jnp.float32),
    )(c0, c1, h)


def _tc_mid(s1, g1, c0, c1, b1, wcat):
    def body(s_ref, g_ref, c0_ref, c1_ref, b_ref, w_ref, o_ref):
        dinv = _dinv_block(c0_ref, c1_ref)
        tot = s_ref[0, :, :_DM] + s_ref[1, :, :_DM] + g_ref[:, :_DM]
        h = jnp.maximum(dinv * tot + b_ref[...], 0.0)
        o_ref[...] = dinv * jnp.dot(h, w_ref[...],
                                    preferred_element_type=jnp.float32)

    return pl.pallas_call(
        body,
        grid=(_N // _BR,),
        in_specs=[pl.BlockSpec((_NC, _BR, _DP), lambda i: (0, i, 0)),
                  pl.BlockSpec((_BR, _DP), lambda i: (i, 0)),
                  pl.BlockSpec((_BR, 1), lambda i: (i, 0)),
                  pl.BlockSpec((_BR, 1), lambda i: (i, 0)),
                  pl.BlockSpec((1, _DM), lambda i: (0, 0)),
                  pl.BlockSpec((_DM, _DP), lambda i: (0, 0))],
        out_specs=pl.BlockSpec((_BR, _DP), lambda i: (i, 0)),
        out_shape=jax.ShapeDtypeStruct((_N, _DP), jnp.float32),
    )(s1, g1, c0, c1, b1, wcat)


def _tc_final(s2, g2, c0, c1, bcat):
    def body(s_ref, g_ref, c0_ref, c1_ref, b_ref, mu_ref, sg_ref):
        tot = s_ref[0, :, :_DC] + s_ref[1, :, :_DC] + g_ref[:, :_DC]
        out = _dinv_block(c0_ref, c1_ref) * tot + b_ref[...]
        mu_ref[...] = out[:, :64]
        sg_ref[...] = out[:, 64:65]

    return pl.pallas_call(
        body,
        grid=(_N // _BR,),
        in_specs=[pl.BlockSpec((_NC, _BR, _DP), lambda i: (0, i, 0)),
                  pl.BlockSpec((_BR, _DP), lambda i: (i, 0)),
                  pl.BlockSpec((_BR, 1), lambda i: (i, 0)),
                  pl.BlockSpec((_BR, 1), lambda i: (i, 0)),
                  pl.BlockSpec((1, _DC), lambda i: (0, 0))],
        out_specs=[pl.BlockSpec((_BR, 64), lambda i: (i, 0)),
                   pl.BlockSpec((_BR, 1), lambda i: (i, 0))],
        out_shape=[jax.ShapeDtypeStruct((_N, 64), jnp.float32),
                   jax.ShapeDtypeStruct((_N, 1), jnp.float32)],
    )(s2, g2, c0, c1, bcat)


def kernel(X, graph, W1, b1, W_mu, b_mu, W_sig, b_sig):
    graph = graph.astype(jnp.int32)
    src4 = graph[0].reshape(_NW, _NSUP, _SUB, _K)
    dst4 = graph[1].reshape(_NW, _NSUP, _SUB, _K)
    wcat = jnp.concatenate(
        [W_mu, W_sig, jnp.zeros((_DM, _DP - 65), jnp.float32)], axis=1)
    bcat = jnp.concatenate(
        [b_mu, b_sig, jnp.zeros((_DC - 65,), jnp.float32)]).reshape(1, _DC)
    b1r = b1.reshape(1, _DM)

    h1 = _tc_matmul(X, W1)
    cnt = _sc_degree(dst4)
    c0 = cnt[:_N].reshape(_N, 1)
    c1 = cnt[_N:].reshape(_N, 1)
    g1 = _tc_scale(c0, c1, h1)
    zrows = jnp.zeros((_N, _DP), jnp.float32)
    s1 = _sc_propagate(src4, dst4, g1, zrows)
    g2 = _tc_mid(s1, g1, c0, c1, b1r, wcat)
    s2 = _sc_propagate(src4, dst4, g2, zrows)
    mus, logsig = _tc_final(s2, g2, c0, c1, bcat)
    return (mus, logsig)


# speedup vs baseline: 1.1619x; 1.0262x over previous
import functools

import jax
import jax.numpy as jnp
from jax import lax
from jax.experimental import pallas as pl
from jax.experimental.pallas import tpu as pltpu
from jax.experimental.pallas import tpu_sc as plsc

_N = 10000
_E = 320000
_IN = 128
_DM = 32
_DC = 80
_DP = 128
_NC, _NS, _L = 2, 16, 16
_NW = _NC * _NS
_ET = _E // _NW
_K = 80
_NCHUNK = _ET // _K
_NSUP = 5
_SUB = _NCHUNK // _NSUP
_NB = 4
_RPT = _N // _NS
_ZR = 25
_BR = 2000


def _mesh():
    return plsc.VectorSubcoreMesh(core_axis_name="c", subcore_axis_name="s")


def _sc_degree(dst4):
    @functools.partial(
        pl.kernel, mesh=_mesh(),
        out_type=jax.ShapeDtypeStruct((_NC * _N,), jnp.float32),
        scratch_types=[
            pltpu.VMEM((_SUB, _K), jnp.int32),
            pltpu.VMEM((_K,), jnp.float32),
            pltpu.VMEM((640,), jnp.float32),
            pltpu.VMEM_SHARED((_N,), jnp.float32),
            pltpu.SemaphoreType.DMA,
        ],
    )
    def deg_kernel(dst_hbm, out_hbm, didx, ones, zbuf, acc, dsem):
        c = lax.axis_index("c")
        s = lax.axis_index("s")
        wid = c * _NS + s

        @pl.loop(0, _K // _L)
        def _fill_ones(i):
            ones[pl.ds(i * _L, _L)] = jnp.full((_L,), 1.0, jnp.float32)

        @pl.loop(0, 640 // _L)
        def _fill_zero(i):
            zbuf[pl.ds(i * _L, _L)] = jnp.zeros((_L,), jnp.float32)

        @pl.loop(s, _N // 640, step=_NS)
        def _zero(j):
            pltpu.sync_copy(zbuf, acc.at[pl.ds(j * 640, 640)])

        @pl.when(s == 0)
        def _zero_tail():
            pltpu.sync_copy(zbuf.at[pl.ds(0, 400)], acc.at[pl.ds(9600, 400)])

        plsc.subcore_barrier()

        @pl.loop(0, _NSUP)
        def _super(u):
            pltpu.sync_copy(dst_hbm.at[wid, u], didx)

            for j in range(_SUB):
                pltpu.async_copy(ones, acc.at[didx.at[j]], dsem, add=True)
            for j in range(_SUB):
                pltpu.make_async_copy(out_hbm.at[pl.ds(0, _K)], ones,
                                      dsem).wait()

        plsc.subcore_barrier()

        @pl.loop(s, _N // 80, step=_NS)
        def _wb(j):
            pltpu.sync_copy(acc.at[pl.ds(j * 80, 80)], zbuf.at[pl.ds(0, 80)])
            pltpu.sync_copy(zbuf.at[pl.ds(0, 80)],
                            out_hbm.at[pl.ds(c * _N + j * 80, 80)])

    return deg_kernel(dst4)


def _sc_propagate(src4, dst4, g, zrows):
    @functools.partial(
        pl.kernel, mesh=_mesh(),
        out_type=jax.ShapeDtypeStruct((_NC, _N, _DP), jnp.float32),
        scratch_types=[
            pltpu.VMEM((_SUB, _K), jnp.int32),
            pltpu.VMEM((_SUB, _K), jnp.int32),
            pltpu.VMEM((_NB, _K, _DP), jnp.float32),
            pltpu.VMEM_SHARED((_N, _DP), jnp.float32),
            pltpu.SemaphoreType.DMA,
            pltpu.SemaphoreType.DMA,
        ],
    )
    def prop_kernel(src_hbm, dst_hbm, g_hbm, z_hbm, out_hbm, sidx, didx,
                    rows, acc, gsem, ssem):
        c = lax.axis_index("c")
        s = lax.axis_index("s")
        wid = c * _NS + s

        def _drain(ref, sem):
            pltpu.make_async_copy(g_hbm.at[pl.ds(0, _K)], ref, sem).wait()

        @pl.loop(s, _N // 80, step=_NS)
        def _zero(j):
            pltpu.sync_copy(z_hbm.at[pl.ds(j * 80, 80)],
                            acc.at[pl.ds(j * 80, 80)])

        plsc.subcore_barrier()

        @pl.loop(0, _NSUP)
        def _super(u):
            pltpu.sync_copy(src_hbm.at[wid, u], sidx)
            pltpu.sync_copy(dst_hbm.at[wid, u], didx)
            for j in range(_NB - 1):
                pltpu.async_copy(g_hbm.at[sidx.at[j]], rows.at[j], gsem)

            for j in range(_SUB):
                _drain(rows.at[j % _NB], gsem)
                if j >= 1:
                    _drain(rows.at[(j - 1) % _NB], ssem)
                if j < _SUB - (_NB - 1):
                    pltpu.async_copy(g_hbm.at[sidx.at[j + _NB - 1]],
                                     rows.at[(j + _NB - 1) % _NB], gsem)
                pltpu.async_copy(rows.at[j % _NB], acc.at[didx.at[j]], ssem,
                                 add=True)

            _drain(rows.at[0], ssem)

        plsc.subcore_barrier()

        @pl.loop(s, _N // 80, step=_NS)
        def _wb(j):
            pltpu.sync_copy(acc.at[pl.ds(j * 80, 80)],
                            out_hbm.at[c, pl.ds(j * 80, 80)])

    return prop_kernel(src4, dst4, g, zrows)


def _dinv_block(c0_ref, c1_ref):
    deg = c0_ref[...] + c1_ref[...] + 1.0
    return lax.rsqrt(deg)


def _tc_scale(c0, c1, x, w):
    def body(c0_ref, c1_ref, x_ref, w_ref, o_ref):
        h = jnp.dot(x_ref[...], w_ref[...], preferred_element_type=jnp.float32)
        g = _dinv_block(c0_ref, c1_ref) * h
        o_ref[...] = jnp.concatenate(
            [g, jnp.zeros((_BR, _DP - _DM), jnp.float32)], axis=1)

    return pl.pallas_call(
        body,
        grid=(_N // _BR,),
        in_specs=[pl.BlockSpec((_BR, 1), lambda i: (i, 0)),
                  pl.BlockSpec((_BR, 1), lambda i: (i, 0)),
                  pl.BlockSpec((_BR, _IN), lambda i: (i, 0)),
                  pl.BlockSpec((_IN, _DM), lambda i: (0, 0))],
        out_specs=pl.BlockSpec((_BR, _DP), lambda i: (i, 0)),
        out_shape=jax.ShapeDtypeStruct((_N, _DP), jnp.float32),
    )(c0, c1, x, w)


def _tc_mid(s1, g1, c0, c1, b1, wcat):
    def body(s_ref, g_ref, c0_ref, c1_ref, b_ref, w_ref, o_ref):
        dinv = _dinv_block(c0_ref, c1_ref)
        tot = s_ref[0, :, :_DM] + s_ref[1, :, :_DM] + g_ref[:, :_DM]
        h = jnp.maximum(dinv * tot + b_ref[...], 0.0)
        o_ref[...] = dinv * jnp.dot(h, w_ref[...],
                                    preferred_element_type=jnp.float32)

    return pl.pallas_call(
        body,
        grid=(_N // _BR,),
        in_specs=[pl.BlockSpec((_NC, _BR, _DP), lambda i: (0, i, 0)),
                  pl.BlockSpec((_BR, _DP), lambda i: (i, 0)),
                  pl.BlockSpec((_BR, 1), lambda i: (i, 0)),
                  pl.BlockSpec((_BR, 1), lambda i: (i, 0)),
                  pl.BlockSpec((1, _DM), lambda i: (0, 0)),
                  pl.BlockSpec((_DM, _DP), lambda i: (0, 0))],
        out_specs=pl.BlockSpec((_BR, _DP), lambda i: (i, 0)),
        out_shape=jax.ShapeDtypeStruct((_N, _DP), jnp.float32),
    )(s1, g1, c0, c1, b1, wcat)


def _tc_final(s2, g2, c0, c1, bcat):
    def body(s_ref, g_ref, c0_ref, c1_ref, b_ref, mu_ref, sg_ref):
        tot = s_ref[0, :, :_DC] + s_ref[1, :, :_DC] + g_ref[:, :_DC]
        out = _dinv_block(c0_ref, c1_ref) * tot + b_ref[...]
        mu_ref[...] = out[:, :64]
        sg_ref[...] = out[:, 64:65]

    return pl.pallas_call(
        body,
        grid=(_N // _BR,),
        in_specs=[pl.BlockSpec((_NC, _BR, _DP), lambda i: (0, i, 0)),
                  pl.BlockSpec((_BR, _DP), lambda i: (i, 0)),
                  pl.BlockSpec((_BR, 1), lambda i: (i, 0)),
                  pl.BlockSpec((_BR, 1), lambda i: (i, 0)),
                  pl.BlockSpec((1, _DC), lambda i: (0, 0))],
        out_specs=[pl.BlockSpec((_BR, 64), lambda i: (i, 0)),
                   pl.BlockSpec((_BR, 1), lambda i: (i, 0))],
        out_shape=[jax.ShapeDtypeStruct((_N, 64), jnp.float32),
                   jax.ShapeDtypeStruct((_N, 1), jnp.float32)],
    )(s2, g2, c0, c1, bcat)


def kernel(X, graph, W1, b1, W_mu, b_mu, W_sig, b_sig):
    graph = graph.astype(jnp.int32)
    src4 = graph[0].reshape(_NW, _NSUP, _SUB, _K)
    dst4 = graph[1].reshape(_NW, _NSUP, _SUB, _K)
    wcat = jnp.concatenate(
        [W_mu, W_sig, jnp.zeros((_DM, _DP - 65), jnp.float32)], axis=1)
    bcat = jnp.concatenate(
        [b_mu, b_sig, jnp.zeros((_DC - 65,), jnp.float32)]).reshape(1, _DC)
    b1r = b1.reshape(1, _DM)

    cnt = _sc_degree(dst4)
    c0 = cnt[:_N].reshape(_N, 1)
    c1 = cnt[_N:].reshape(_N, 1)
    g1 = _tc_scale(c0, c1, X, W1)
    zrows = jnp.zeros((_N, _DP), jnp.float32)
    s1 = _sc_propagate(src4, dst4, g1, zrows)
    g2 = _tc_mid(s1, g1, c0, c1, b1r, wcat)
    s2 = _sc_propagate(src4, dst4, g2, zrows)
    mus, logsig = _tc_final(s2, g2, c0, c1, bcat)
    return (mus, logsig)


# speedup vs baseline: 1.1790x; 1.0146x over previous
import functools

import jax
import jax.numpy as jnp
from jax import lax
from jax.experimental import pallas as pl
from jax.experimental.pallas import tpu as pltpu
from jax.experimental.pallas import tpu_sc as plsc

_N = 10000
_E = 320000
_IN = 128
_DM = 32
_DC = 80
_DP = 128
_NC, _NS, _L = 2, 16, 16
_NW = _NC * _NS
_ET = _E // _NW
_K = 80
_NCHUNK = _ET // _K
_NSUP = 5
_SUB = _NCHUNK // _NSUP
_NB = 4
_RPT = _N // _NS
_ZR = 25
_BR = 2000


def _mesh():
    return plsc.VectorSubcoreMesh(core_axis_name="c", subcore_axis_name="s")


def _sc_degree(dst4):
    @functools.partial(
        pl.kernel, mesh=_mesh(),
        out_type=jax.ShapeDtypeStruct((_NC * _N,), jnp.float32),
        scratch_types=[
            pltpu.VMEM((_SUB, _K), jnp.int32),
            pltpu.VMEM((_K,), jnp.float32),
            pltpu.VMEM((640,), jnp.float32),
            pltpu.VMEM_SHARED((_N,), jnp.float32),
            pltpu.SemaphoreType.DMA,
        ],
    )
    def deg_kernel(dst_hbm, out_hbm, didx, ones, zbuf, acc, dsem):
        c = lax.axis_index("c")
        s = lax.axis_index("s")
        wid = c * _NS + s

        @pl.loop(0, _K // _L)
        def _fill_ones(i):
            ones[pl.ds(i * _L, _L)] = jnp.full((_L,), 1.0, jnp.float32)

        @pl.loop(0, 640 // _L)
        def _fill_zero(i):
            zbuf[pl.ds(i * _L, _L)] = jnp.zeros((_L,), jnp.float32)

        @pl.loop(s, _N // 640, step=_NS)
        def _zero(j):
            pltpu.sync_copy(zbuf, acc.at[pl.ds(j * 640, 640)])

        @pl.when(s == 0)
        def _zero_tail():
            pltpu.sync_copy(zbuf.at[pl.ds(0, 400)], acc.at[pl.ds(9600, 400)])

        plsc.subcore_barrier()

        @pl.loop(0, _NSUP)
        def _super(u):
            pltpu.sync_copy(dst_hbm.at[wid, u], didx)

            for j in range(_SUB):
                pltpu.async_copy(ones, acc.at[didx.at[j]], dsem, add=True)
            for j in range(_SUB):
                pltpu.make_async_copy(out_hbm.at[pl.ds(0, _K)], ones,
                                      dsem).wait()

        plsc.subcore_barrier()

        @pl.loop(s, _N // 80, step=_NS)
        def _wb(j):
            pltpu.sync_copy(acc.at[pl.ds(j * 80, 80)], zbuf.at[pl.ds(0, 80)])
            pltpu.sync_copy(zbuf.at[pl.ds(0, 80)],
                            out_hbm.at[pl.ds(c * _N + j * 80, 80)])

    return deg_kernel(dst4)


def _sc_propagate(src4, dst4, g, zrows):
    @functools.partial(
        pl.kernel, mesh=_mesh(),
        out_type=jax.ShapeDtypeStruct((_NC, _N, _DP), jnp.float32),
        scratch_types=[
            pltpu.VMEM((_SUB, _K), jnp.int32),
            pltpu.VMEM((_SUB, _K), jnp.int32),
            pltpu.VMEM((_NB, _K, _DP), jnp.float32),
            pltpu.VMEM_SHARED((_N, _DP), jnp.float32),
            pltpu.SemaphoreType.DMA,
            pltpu.SemaphoreType.DMA,
        ],
    )
    def prop_kernel(src_hbm, dst_hbm, g_hbm, z_hbm, out_hbm, sidx, didx,
                    rows, acc, gsem, ssem):
        c = lax.axis_index("c")
        s = lax.axis_index("s")
        wid = c * _NS + s

        def _drain(ref, sem):
            pltpu.make_async_copy(g_hbm.at[pl.ds(0, _K)], ref, sem).wait()

        @pl.loop(s, _N // 80, step=_NS)
        def _zero(j):
            pltpu.async_copy(z_hbm.at[pl.ds(j * 80, 80)],
                             acc.at[pl.ds(j * 80, 80)], gsem)

        @pl.loop(s, _N // 80, step=_NS)
        def _zero_drain(j):
            pltpu.make_async_copy(z_hbm.at[pl.ds(0, 80)],
                                  acc.at[pl.ds(j * 80, 80)], gsem).wait()

        plsc.subcore_barrier()

        @pl.loop(0, _NSUP)
        def _super(u):
            pltpu.sync_copy(src_hbm.at[wid, u], sidx)
            pltpu.sync_copy(dst_hbm.at[wid, u], didx)
            for j in range(_NB - 1):
                pltpu.async_copy(g_hbm.at[sidx.at[j]], rows.at[j], gsem)

            for j in range(_SUB):
                _drain(rows.at[j % _NB], gsem)
                if j >= 1:
                    _drain(rows.at[(j - 1) % _NB], ssem)
                if j < _SUB - (_NB - 1):
                    pltpu.async_copy(g_hbm.at[sidx.at[j + _NB - 1]],
                                     rows.at[(j + _NB - 1) % _NB], gsem)
                pltpu.async_copy(rows.at[j % _NB], acc.at[didx.at[j]], ssem,
                                 add=True)

            _drain(rows.at[0], ssem)

        plsc.subcore_barrier()

        @pl.loop(s, _N // 80, step=_NS)
        def _wb(j):
            pltpu.async_copy(acc.at[pl.ds(j * 80, 80)],
                             out_hbm.at[c, pl.ds(j * 80, 80)], gsem)

        @pl.loop(s, _N // 80, step=_NS)
        def _wb_drain(j):
            pltpu.make_async_copy(z_hbm.at[pl.ds(0, 80)],
                                  acc.at[pl.ds(j * 80, 80)], gsem).wait()

    return prop_kernel(src4, dst4, g, zrows)


def _dinv_block(c0_ref, c1_ref):
    deg = c0_ref[...] + c1_ref[...] + 1.0
    return lax.rsqrt(deg)


def _tc_scale(c0, c1, x, w):
    def body(c0_ref, c1_ref, x_ref, w_ref, o_ref):
        h = jnp.dot(x_ref[...], w_ref[...], preferred_element_type=jnp.float32)
        g = _dinv_block(c0_ref, c1_ref) * h
        o_ref[...] = jnp.concatenate(
            [g, jnp.zeros((_BR, _DP - _DM), jnp.float32)], axis=1)

    return pl.pallas_call(
        body,
        grid=(_N // _BR,),
        in_specs=[pl.BlockSpec((_BR, 1), lambda i: (i, 0)),
                  pl.BlockSpec((_BR, 1), lambda i: (i, 0)),
                  pl.BlockSpec((_BR, _IN), lambda i: (i, 0)),
                  pl.BlockSpec((_IN, _DM), lambda i: (0, 0))],
        out_specs=pl.BlockSpec((_BR, _DP), lambda i: (i, 0)),
        out_shape=jax.ShapeDtypeStruct((_N, _DP), jnp.float32),
    )(c0, c1, x, w)


def _tc_mid(s1, g1, c0, c1, b1, wcat):
    def body(s_ref, g_ref, c0_ref, c1_ref, b_ref, w_ref, o_ref):
        dinv = _dinv_block(c0_ref, c1_ref)
        tot = s_ref[0, :, :_DM] + s_ref[1, :, :_DM] + g_ref[:, :_DM]
        h = jnp.maximum(dinv * tot + b_ref[...], 0.0)
        o_ref[...] = dinv * jnp.dot(h, w_ref[...],
                                    preferred_element_type=jnp.float32)

    return pl.pallas_call(
        body,
        grid=(_N // _BR,),
        in_specs=[pl.BlockSpec((_NC, _BR, _DP), lambda i: (0, i, 0)),
                  pl.BlockSpec((_BR, _DP), lambda i: (i, 0)),
                  pl.BlockSpec((_BR, 1), lambda i: (i, 0)),
                  pl.BlockSpec((_BR, 1), lambda i: (i, 0)),
                  pl.BlockSpec((1, _DM), lambda i: (0, 0)),
                  pl.BlockSpec((_DM, _DP), lambda i: (0, 0))],
        out_specs=pl.BlockSpec((_BR, _DP), lambda i: (i, 0)),
        out_shape=jax.ShapeDtypeStruct((_N, _DP), jnp.float32),
    )(s1, g1, c0, c1, b1, wcat)


def _tc_final(s2, g2, c0, c1, bcat):
    def body(s_ref, g_ref, c0_ref, c1_ref, b_ref, mu_ref, sg_ref):
        tot = s_ref[0, :, :_DC] + s_ref[1, :, :_DC] + g_ref[:, :_DC]
        out = _dinv_block(c0_ref, c1_ref) * tot + b_ref[...]
        mu_ref[...] = out[:, :64]
        sg_ref[...] = out[:, 64:65]

    return pl.pallas_call(
        body,
        grid=(_N // _BR,),
        in_specs=[pl.BlockSpec((_NC, _BR, _DP), lambda i: (0, i, 0)),
                  pl.BlockSpec((_BR, _DP), lambda i: (i, 0)),
                  pl.BlockSpec((_BR, 1), lambda i: (i, 0)),
                  pl.BlockSpec((_BR, 1), lambda i: (i, 0)),
                  pl.BlockSpec((1, _DC), lambda i: (0, 0))],
        out_specs=[pl.BlockSpec((_BR, 64), lambda i: (i, 0)),
                   pl.BlockSpec((_BR, 1), lambda i: (i, 0))],
        out_shape=[jax.ShapeDtypeStruct((_N, 64), jnp.float32),
                   jax.ShapeDtypeStruct((_N, 1), jnp.float32)],
    )(s2, g2, c0, c1, bcat)


def kernel(X, graph, W1, b1, W_mu, b_mu, W_sig, b_sig):
    graph = graph.astype(jnp.int32)
    src4 = graph[0].reshape(_NW, _NSUP, _SUB, _K)
    dst4 = graph[1].reshape(_NW, _NSUP, _SUB, _K)
    wcat = jnp.concatenate(
        [W_mu, W_sig, jnp.zeros((_DM, _DP - 65), jnp.float32)], axis=1)
    bcat = jnp.concatenate(
        [b_mu, b_sig, jnp.zeros((_DC - 65,), jnp.float32)]).reshape(1, _DC)
    b1r = b1.reshape(1, _DM)

    cnt = _sc_degree(dst4)
    c0 = cnt[:_N].reshape(_N, 1)
    c1 = cnt[_N:].reshape(_N, 1)
    g1 = _tc_scale(c0, c1, X, W1)
    zrows = jnp.zeros((_N, _DP), jnp.float32)
    s1 = _sc_propagate(src4, dst4, g1, zrows)
    g2 = _tc_mid(s1, g1, c0, c1, b1r, wcat)
    s2 = _sc_propagate(src4, dst4, g2, zrows)
    mus, logsig = _tc_final(s2, g2, c0, c1, bcat)
    return (mus, logsig)
